# baseline (speedup 1.0000x reference)
import functools

import jax
import jax.numpy as jnp
from jax import lax
from jax.experimental import pallas as pl
from jax.experimental.pallas import tpu as pltpu
from jax.experimental.pallas import tpu_sc as plsc

VOCAB = 1000000
D = 64
SEQ = 200
BATCH = 1024
NC = 2
NS = 16
NW = NC * NS
ROWS = BATCH * SEQ
ROWS_PER_W = ROWS // NW
SEQ_PER_W = ROWS_PER_W // SEQ
HALF = SEQ // 2


def _positional_encoding(seq_len, dim):
    pos = jnp.arange(seq_len, dtype=jnp.float32)[:, None]
    half_idx = jnp.arange(dim // 2, dtype=jnp.float32)
    rates = jnp.power(10000.0, -2.0 * half_idx / float(dim))
    ang = pos * rates[None, :]
    pe = jnp.stack([jnp.sin(ang), jnp.cos(ang)], axis=-1)
    return pe.reshape(seq_len, dim)


_mesh = plsc.VectorSubcoreMesh(core_axis_name="c", subcore_axis_name="s")


@functools.partial(
    pl.kernel,
    out_type=jax.ShapeDtypeStruct((ROWS, D), jnp.float32),
    mesh=_mesh,
    scratch_types=[
        pltpu.VMEM((2 * SEQ_PER_W, HALF), jnp.int32),
        pltpu.VMEM((SEQ, D), jnp.float32),
        pltpu.VMEM_SHARED((SEQ, D), jnp.float32),
        pltpu.SemaphoreType.DMA,
    ],
    compiler_params=pltpu.CompilerParams(use_tc_tiling_on_sc=False),
)
def _sc_embed(idx_hbm, pe_hbm, table_hbm, out_hbm, idx_v, rows_v, pe_sh, sem):
    cid = lax.axis_index("c")
    sid = lax.axis_index("s")
    wid = sid * NC + cid
    base = wid * ROWS_PER_W

    @pl.when(sid == 0)
    def _():
        pltpu.sync_copy(pe_hbm, pe_sh)

    plsc.subcore_barrier()

    pltpu.sync_copy(idx_hbm.at[wid], idx_v)

    def body(s, carry):
        pltpu.sync_copy(pe_sh, rows_v)
        pltpu.async_copy(
            table_hbm.at[idx_v.at[2 * s]], rows_v.at[pl.ds(0, HALF)], sem,
            add=True).wait()
        pltpu.async_copy(
            table_hbm.at[idx_v.at[2 * s + 1]], rows_v.at[pl.ds(HALF, HALF)],
            sem, add=True).wait()
        pltpu.sync_copy(rows_v, out_hbm.at[pl.ds(base + s * SEQ, SEQ)])
        return carry

    lax.fori_loop(0, SEQ_PER_W, body, 0)


def kernel(x, table):
    idx = x.reshape(NW, 2 * SEQ_PER_W, HALF).astype(jnp.int32)
    pe = _positional_encoding(SEQ, D)
    out = _sc_embed(idx, pe, table)
    return out.reshape(BATCH, SEQ, D)


# speedup vs baseline: 1.0421x; 1.0421x over previous
import functools

import jax
import jax.numpy as jnp
from jax import lax
from jax.experimental import pallas as pl
from jax.experimental.pallas import tpu as pltpu
from jax.experimental.pallas import tpu_sc as plsc

VOCAB = 1000000
D = 64
SEQ = 200
BATCH = 1024
NC = 2
NS = 16
NW = NC * NS
SEQ_PER_W = BATCH // NW
HALF = SEQ // 2
PAIRS = SEQ_PER_W // 2


def _positional_encoding(seq_len, dim):
    pos = jnp.arange(seq_len, dtype=jnp.float32)[:, None]
    half_idx = jnp.arange(dim // 2, dtype=jnp.float32)
    rates = jnp.power(10000.0, -2.0 * half_idx / float(dim))
    ang = pos * rates[None, :]
    pe = jnp.stack([jnp.sin(ang), jnp.cos(ang)], axis=-1)
    return pe.reshape(seq_len, dim)


_mesh = plsc.VectorSubcoreMesh(core_axis_name="c", subcore_axis_name="s")


@functools.partial(
    pl.kernel,
    out_type=jax.ShapeDtypeStruct((BATCH, SEQ, D), jnp.float32),
    mesh=_mesh,
    scratch_types=[
        pltpu.VMEM((2 * SEQ_PER_W, HALF), jnp.int32),
        pltpu.VMEM((SEQ, D), jnp.float32),
        pltpu.VMEM((SEQ, D), jnp.float32),
        pltpu.VMEM_SHARED((SEQ, D), jnp.float32),
        pltpu.SemaphoreType.DMA((2,)),
        pltpu.SemaphoreType.DMA((2,)),
        pltpu.SemaphoreType.DMA,
    ],
    compiler_params=pltpu.CompilerParams(use_tc_tiling_on_sc=False),
)
def _sc_embed(idx_hbm, pe_hbm, table_hbm, out_hbm,
              idx_v, rows_a, rows_b, pe_sh, sem_i, sem_w, sem_g):
    cid = lax.axis_index("c")
    sid = lax.axis_index("s")
    wid = sid * NC + cid
    seq0 = wid * SEQ_PER_W

    @pl.when(sid == 0)
    def _():
        pltpu.sync_copy(pe_hbm, pe_sh)

    plsc.subcore_barrier()

    pltpu.sync_copy(idx_hbm.at[wid], idx_v)
    pltpu.async_copy(pe_sh, rows_a, sem_i.at[0])

    def gather_seq(rows, s):
        c1 = pltpu.async_copy(
            table_hbm.at[idx_v.at[2 * s]], rows.at[pl.ds(0, HALF)], sem_g,
            add=True)
        c2 = pltpu.async_copy(
            table_hbm.at[idx_v.at[2 * s + 1]], rows.at[pl.ds(HALF, HALF)],
            sem_g, add=True)
        c1.wait()
        c2.wait()

    def body(g, carry):
        s0 = 2 * g
        s1 = 2 * g + 1

        @pl.when(g > 0)
        def _():
            pltpu.make_async_copy(rows_b, out_hbm.at[0], sem_w.at[1]).wait()
        pltpu.async_copy(pe_sh, rows_b, sem_i.at[1])

        pltpu.make_async_copy(pe_sh, rows_a, sem_i.at[0]).wait()
        gather_seq(rows_a, s0)
        pltpu.async_copy(rows_a, out_hbm.at[seq0 + s0], sem_w.at[0])

        pltpu.make_async_copy(pe_sh, rows_b, sem_i.at[1]).wait()
        gather_seq(rows_b, s1)
        pltpu.async_copy(rows_b, out_hbm.at[seq0 + s1], sem_w.at[1])

        @pl.when(g < PAIRS - 1)
        def _():
            pltpu.make_async_copy(rows_a, out_hbm.at[0], sem_w.at[0]).wait()
            pltpu.async_copy(pe_sh, rows_a, sem_i.at[0])

        return carry

    lax.fori_loop(0, PAIRS, body, 0)
    pltpu.make_async_copy(rows_a, out_hbm.at[0], sem_w.at[0]).wait()
    pltpu.make_async_copy(rows_b, out_hbm.at[0], sem_w.at[1]).wait()


def kernel(x, table):
    idx = x.reshape(NW, 2 * SEQ_PER_W, HALF).astype(jnp.int32)
    pe = _positional_encoding(SEQ, D)
    return _sc_embed(idx, pe, table)


# speedup vs baseline: 1.2811x; 1.2293x over previous
import functools
import jax
import jax.numpy as jnp
from jax import lax
from jax.experimental import pallas as pl
from jax.experimental.pallas import tpu as pltpu
from jax.experimental.pallas import tpu_sc as plsc

VOCAB = 1000000
D = 64
SEQ = 200
BATCH = 1024
NC, NS = 2, 16
NW = NC * NS
SEQ_PER_W = BATCH // NW
GROUP_STARTS = tuple(list(range(0, SEQ - 16, 16)) + [SEQ - 16])

_mesh = plsc.VectorSubcoreMesh(core_axis_name="c", subcore_axis_name="s")


@functools.partial(
    pl.kernel,
    out_type=jax.ShapeDtypeStruct((BATCH, SEQ, D), jnp.float32),
    mesh=_mesh,
    scratch_types=[
        pltpu.VMEM((SEQ_PER_W, SEQ), jnp.int32),
        pltpu.VMEM((SEQ, D), jnp.float32),
        pltpu.VMEM((SEQ, D), jnp.float32),
        pltpu.SemaphoreType.DMA,
        pltpu.SemaphoreType.DMA,
    ],
)
def _sc_embed(x_hbm, pe_hbm, table_hbm, out_hbm,
              idx_v, rows_v, pe_v, sem_g, sem_w):
    cid = lax.axis_index("c")
    sid = lax.axis_index("s")
    wid = sid * NC + cid
    seq0 = wid * SEQ_PER_W

    pltpu.sync_copy(pe_hbm, pe_v)
    pltpu.sync_copy(x_hbm.at[pl.ds(seq0, SEQ_PER_W)], idx_v)

    def body(s, carry):
        for base in GROUP_STARTS:
            vec = idx_v[s, pl.ds(base, 16)]
            for j in range(16):
                r = vec[j]
                pltpu.async_copy(table_hbm.at[r], rows_v.at[base + j], sem_g)

        def drain(j, c):
            pltpu.make_async_copy(table_hbm.at[0], rows_v.at[0], sem_g).wait()
            return c

        lax.fori_loop(0, 16 * len(GROUP_STARTS), drain, 0)

        def vadd(i, c):
            q = i // 4
            d = i % 4
            rows_v[q, pl.ds(d * 16, 16)] = (
                rows_v[q, pl.ds(d * 16, 16)] + pe_v[q, pl.ds(d * 16, 16)])
            return c

        lax.fori_loop(0, SEQ * 4, vadd, 0)
        pltpu.async_copy(rows_v, out_hbm.at[seq0 + s], sem_w)
        pltpu.make_async_copy(rows_v, out_hbm.at[0], sem_w).wait()
        return carry

    lax.fori_loop(0, SEQ_PER_W, body, 0)


def _positional_encoding(seq_len, dim):
    pos = jnp.arange(seq_len, dtype=jnp.float32)[:, None]
    half_idx = jnp.arange(dim // 2, dtype=jnp.float32)
    rates = jnp.power(10000.0, -2.0 * half_idx / float(dim))
    ang = pos * rates[None, :]
    pe = jnp.stack([jnp.sin(ang), jnp.cos(ang)], axis=-1)
    return pe.reshape(seq_len, dim)


def kernel(x, table):
    pe = _positional_encoding(SEQ, D)
    return _sc_embed(x.astype(jnp.int32), pe, table)


# speedup vs baseline: 1.3921x; 1.0867x over previous
import functools

import jax
import jax.numpy as jnp
from jax import lax
from jax.experimental import pallas as pl
from jax.experimental.pallas import tpu as pltpu
from jax.experimental.pallas import tpu_sc as plsc

VOCAB = 1000000
D = 64
SEQ = 200
BATCH = 1024
NC, NS = 2, 16
NW = NC * NS
SEQ_PER_W = BATCH // NW
NBUF = 3
GROUP_STARTS = tuple(range(0, SEQ - 16, 16)) + (SEQ - 16,)

_mesh = plsc.VectorSubcoreMesh(core_axis_name="c", subcore_axis_name="s")


def _positional_encoding(seq_len, dim):
    pos = jnp.arange(seq_len, dtype=jnp.float32)[:, None]
    half_idx = jnp.arange(dim // 2, dtype=jnp.float32)
    rates = jnp.power(10000.0, -2.0 * half_idx / float(dim))
    ang = pos * rates[None, :]
    pe = jnp.stack([jnp.sin(ang), jnp.cos(ang)], axis=-1)
    return pe.reshape(seq_len, dim)


@functools.partial(
    pl.kernel,
    out_type=jax.ShapeDtypeStruct((BATCH, SEQ, D), jnp.float32),
    mesh=_mesh,
    scratch_types=[
        pltpu.VMEM((SEQ_PER_W, SEQ), jnp.int32),
        pltpu.VMEM((NBUF, SEQ, D), jnp.float32),
        pltpu.VMEM((SEQ, D), jnp.float32),
        pltpu.SemaphoreType.DMA((NBUF,)),
        pltpu.SemaphoreType.DMA((NBUF,)),
    ],
)
def _sc_embed(x_hbm, pe_hbm, table_hbm, out_hbm,
              idx_v, rows_v, pe_v, sem_g, sem_w):
    cid = lax.axis_index("c")
    sid = lax.axis_index("s")
    wid = sid * NC + cid
    seq0 = wid * SEQ_PER_W

    pltpu.sync_copy(pe_hbm, pe_v)
    pltpu.sync_copy(x_hbm.at[pl.ds(seq0, SEQ_PER_W)], idx_v)

    def issue_gathers(s, b):
        for base in GROUP_STARTS:
            vec = idx_v[s, pl.ds(base, 16)]
            lanes = range(8, 16) if base == SEQ - 16 else range(16)
            for j in lanes:
                r = vec[j]
                pltpu.async_copy(
                    table_hbm.at[r], rows_v.at[b, base + j], sem_g.at[b])

    issue_gathers(0, 0)

    def phase(s, carry):
        b = lax.rem(s, NBUF)
        nb = lax.rem(s + 1, NBUF)

        @pl.when(s < SEQ_PER_W - 1)
        def _():
            @pl.when(s >= NBUF - 1)
            def _():
                pltpu.make_async_copy(
                    rows_v.at[nb], out_hbm.at[0], sem_w.at[nb]).wait()
            issue_gathers(s + 1, nb)

        pltpu.make_async_copy(
            out_hbm.at[0], rows_v.at[b], sem_g.at[b]).wait()

        def vadd(q, c):
            for d in range(4):
                sl = pl.ds(d * 16, 16)
                rows_v[b, q, sl] = rows_v[b, q, sl] + pe_v[q, sl]
            return c

        lax.fori_loop(0, SEQ, vadd, 0)

        pltpu.async_copy(rows_v.at[b], out_hbm.at[seq0 + s], sem_w.at[b])
        return carry

    lax.fori_loop(0, SEQ_PER_W, phase, 0)

    def final_drain(k, c):
        b = lax.rem(SEQ_PER_W - 1 - k, NBUF)
        pltpu.make_async_copy(rows_v.at[b], out_hbm.at[0], sem_w.at[b]).wait()
        return c

    lax.fori_loop(0, min(NBUF, SEQ_PER_W), final_drain, 0)


def kernel(x, table):
    pe = _positional_encoding(SEQ, D)
    return _sc_embed(x.astype(jnp.int32), pe, table)
